# baseline (speedup 1.0000x reference)
import jax
import jax.numpy as jnp
from jax import lax
from jax.experimental import pallas as pl
from jax.experimental.pallas import tpu as pltpu

SEARCH = 8
TB = 128
G = 4


def _body(u_t_ref, x_ref, Wp_ref, bp_ref, Ws_ref, Wt_ref, out_ref):
    x = x_ref[...]
    Wp = Wp_ref[...]
    bp = bp_ref[...]
    Ws = Ws_ref[...]
    Wt = Wt_ref[...]
    DIN, H = Wp.shape
    Nj = x.shape[1]

    zc = jnp.zeros((DIN, H), jnp.float32)
    W4 = jnp.concatenate(
        [jnp.concatenate([Wp if p == q else zc for q in range(G)], axis=1)
         for p in range(G)], axis=0)
    zt = jnp.zeros((H, H), jnp.float32)
    Wt4 = jnp.concatenate(
        [jnp.concatenate([Wt if p == q else zt for q in range(G)], axis=1)
         for p in range(G)], axis=0)
    bp4 = jnp.concatenate([bp] * G, axis=0)

    ic4 = lax.dot_general(W4, x, (((0,), (2,)), ((), ())),
                          preferred_element_type=jnp.float32)
    ic4 = ic4 + bp4[:, :, None]
    tgt4 = jnp.tanh(lax.dot_general(Wt4, ic4, (((0,), (0,)), ((), ())),
                                    preferred_element_type=jnp.float32))

    ic_a = jnp.swapaxes(ic4, 1, 2).reshape(G, H, Nj, TB)
    tgt_a = jnp.swapaxes(tgt4, 1, 2).reshape(G, H, Nj, TB)

    ssum = u_t_ref[...]
    out_ref[0, :, :] = ssum
    n_iota = (G * lax.broadcasted_iota(jnp.int32, (G, Nj, TB), 1)
              + lax.broadcasted_iota(jnp.int32, (G, Nj, TB), 0))
    for i in range(SEARCH):
        m = ssum * (1.0 / (i + 1.0))
        src = jnp.tanh(lax.dot_general(Ws, m, (((0,), (0,)), ((), ())),
                                       preferred_element_type=jnp.float32))
        scores = jnp.sum(tgt_a * src[None, :, None, :], axis=1)
        mx = jnp.max(scores, axis=(0, 1), keepdims=True)
        cand = jnp.where(scores == mx, n_iota, jnp.int32(2**30))
        idx = jnp.min(cand, axis=(0, 1), keepdims=True)
        onehot = (n_iota == idx).astype(jnp.float32)
        item = jnp.sum(ic_a * onehot[:, None, :, :], axis=(0, 2))
        ssum = ssum + item
        out_ref[i + 1, :, :] = item


def kernel(user_intent, item_corpus, W_proj, b_proj, W_s, W_t):
    B, N, DIN = item_corpus.shape
    H = W_proj.shape[1]
    Nj = N // G
    xr = item_corpus.reshape(B, Nj, G * DIN)
    grid = (B // TB,)
    out = pl.pallas_call(
        _body,
        grid=grid,
        in_specs=[
            pl.BlockSpec((H, TB), lambda g: (0, g)),
            pl.BlockSpec((TB, Nj, G * DIN), lambda g: (g, 0, 0)),
            pl.BlockSpec((DIN, H), lambda g: (0, 0)),
            pl.BlockSpec((H, 1), lambda g: (0, 0)),
            pl.BlockSpec((H, H), lambda g: (0, 0)),
            pl.BlockSpec((H, H), lambda g: (0, 0)),
        ],
        out_specs=pl.BlockSpec((SEARCH + 1, H, TB), lambda g: (0, 0, g)),
        out_shape=jax.ShapeDtypeStruct((SEARCH + 1, H, B), jnp.float32),
    )(user_intent.T, xr, W_proj, b_proj.reshape(H, 1), W_s, W_t)
    return jnp.transpose(out, (2, 0, 1))
